# baseline (speedup 1.0000x reference)
import functools

import jax
import jax.numpy as jnp
from jax import lax
from jax.experimental import pallas as pl
from jax.experimental.pallas import tpu as pltpu
from jax.experimental.pallas import tpu_sc as plsc

D = 64
EPS = 1e-5
NC, NS, LANES = 2, 16, 16
NW = NC * NS


def _ln_gather_body(bpw, table, idx, gamma, beta, out,
                    idx_v, rows_v, gam_v, bet_v, sem):
    wid = lax.axis_index("s") * NC + lax.axis_index("c")
    base = wid * bpw

    pltpu.sync_copy(idx.at[pl.ds(base, bpw)], idx_v)
    pltpu.sync_copy(gamma, gam_v)
    pltpu.sync_copy(beta, bet_v)
    pltpu.async_copy(table.at[idx_v], rows_v, sem).wait()

    g = [gam_v[pl.ds(j * LANES, LANES)] for j in range(D // LANES)]
    b = [bet_v[pl.ds(j * LANES, LANES)] for j in range(D // LANES)]
    inv_d = jnp.float32(1.0 / D)
    iota = lax.iota(jnp.int32, LANES)

    gdn = lax.GatherDimensionNumbers(
        offset_dims=(), collapsed_slice_dims=(0,), start_index_map=(0,))

    def shuffle(x, p):
        return lax.gather(x, p[:, None], dimension_numbers=gdn,
                          slice_sizes=(1,),
                          mode=lax.GatherScatterMode.PROMISE_IN_BOUNDS)

    def rsqrt_v(xv):
        iv = lax.bitcast_convert_type(xv, jnp.int32)
        iv = jnp.int32(0x5F3759DF) - lax.shift_right_logical(iv, 1)
        y = lax.bitcast_convert_type(iv, jnp.float32)
        half_x = xv * jnp.float32(0.5)
        for _ in range(3):
            y = y * (jnp.float32(1.5) - half_x * y * y)
        return y

    csplat = [jnp.full((LANES,), c, dtype=jnp.int32) for c in range(D)]
    rsplat = [jnp.full((LANES,), r, dtype=jnp.int32) for r in range(LANES)]

    def group_body(gi, carry):
        rbase = gi * LANES
        row_ids = rbase + iota
        acc = jnp.zeros((LANES,), jnp.float32)
        acc2 = jnp.zeros((LANES,), jnp.float32)
        for c in range(D):
            v = plsc.load_gather(rows_v, [row_ids, csplat[c]])
            acc = acc + v
            acc2 = acc2 + v * v
        mean = acc * inv_d
        var = acc2 * inv_d - mean * mean
        y = rsqrt_v(var + EPS)
        for r in range(LANES):
            m = shuffle(mean, rsplat[r])
            s = shuffle(y, rsplat[r])
            row = rbase + r
            for j in range(D // LANES):
                v = rows_v[row, pl.ds(j * LANES, LANES)]
                rows_v[row, pl.ds(j * LANES, LANES)] = (v - m) * s * g[j] + b[j]
        return carry

    lax.fori_loop(0, bpw // LANES, group_body, 0)

    pltpu.sync_copy(rows_v, out.at[pl.ds(base, bpw)])


def kernel(layer_indices, layer_type, L, device, emb_table, ln_gamma, ln_beta):
    del layer_type, device
    n = layer_indices.shape[0]
    assert n % NW == 0
    bpw = n // NW
    idx32 = layer_indices.astype(jnp.int32)

    mesh = plsc.VectorSubcoreMesh(core_axis_name="c", subcore_axis_name="s")
    run = pl.kernel(
        functools.partial(_ln_gather_body, bpw),
        mesh=mesh,
        out_type=jax.ShapeDtypeStruct((n, D), jnp.float32),
        scratch_types=[
            pltpu.VMEM((bpw,), jnp.int32),
            pltpu.VMEM((bpw, D), jnp.float32),
            pltpu.VMEM((D,), jnp.float32),
            pltpu.VMEM((D,), jnp.float32),
            pltpu.SemaphoreType.DMA,
        ],
        compiler_params=pltpu.CompilerParams(
            use_tc_tiling_on_sc=False, needs_layout_passes=False),
    )
    return run(emb_table, idx32, ln_gamma, ln_beta)


# speedup vs baseline: 1.9083x; 1.9083x over previous
import functools

import jax
import jax.numpy as jnp
from jax import lax
from jax.experimental import pallas as pl
from jax.experimental.pallas import tpu as pltpu
from jax.experimental.pallas import tpu_sc as plsc

D = 64
EPS = 1e-5
NC, NS, LANES = 2, 16, 16
NW = NC * NS
CH = 32


def _ln_gather_body(bpw, table3, idx, gamma, beta, out,
                    idx_v, rows_v, gam_v, bet_v, sem):
    wid = lax.axis_index("s") * NC + lax.axis_index("c")
    base = wid * bpw

    pltpu.sync_copy(idx.at[pl.ds(base, bpw)], idx_v)
    pltpu.sync_copy(gamma, gam_v)
    pltpu.sync_copy(beta, bet_v)

    def issue_body(gi, carry):
        vb = gi * LANES
        t = idx_v[pl.ds(vb, LANES)]
        gv = lax.shift_right_logical(t, 3)
        sv = lax.bitwise_and(t, 7)
        for k in range(LANES):
            pltpu.async_copy(table3.at[gv[k], sv[k]], rows_v.at[vb + k], sem)
        return carry

    lax.fori_loop(0, bpw // LANES, issue_body, 0)

    g = [gam_v[pl.ds(j * LANES, LANES)] for j in range(D // LANES)]
    b = [bet_v[pl.ds(j * LANES, LANES)] for j in range(D // LANES)]
    inv_d = jnp.float32(1.0 / D)
    iota = lax.iota(jnp.int32, LANES)
    perms = [lax.bitwise_xor(iota, jnp.int32(sh)) for sh in (8, 4, 2, 1)]

    gdn = lax.GatherDimensionNumbers(
        offset_dims=(), collapsed_slice_dims=(0,), start_index_map=(0,))

    def shuffle(x, p):
        return lax.gather(x, p[:, None], dimension_numbers=gdn,
                          slice_sizes=(1,),
                          mode=lax.GatherScatterMode.PROMISE_IN_BOUNDS)

    def allsum(x):
        for p in perms:
            x = x + shuffle(x, p)
        return x

    def ln_row(r):
        v = [rows_v[r, pl.ds(j * LANES, LANES)] for j in range(D // LANES)]
        s = (v[0] + v[1]) + (v[2] + v[3])
        sq = (v[0] * v[0] + v[1] * v[1]) + (v[2] * v[2] + v[3] * v[3])
        mean = allsum(s) * inv_d
        var = allsum(sq) * inv_d - mean * mean
        xv = var + EPS
        iv = lax.bitcast_convert_type(xv, jnp.int32)
        iv = jnp.int32(0x5F3759DF) - lax.shift_right_logical(iv, 1)
        y = lax.bitcast_convert_type(iv, jnp.float32)
        half_x = xv * jnp.float32(0.5)
        for _ in range(3):
            y = y * (jnp.float32(1.5) - half_x * y * y)
        for j in range(D // LANES):
            rows_v[r, pl.ds(j * LANES, LANES)] = (v[j] - mean) * y * g[j] + b[j]

    def chunk_body(c, carry):
        cb = c * CH
        for k in range(CH):
            pltpu.make_async_copy(
                table3.at[0, 0], rows_v.at[cb + k], sem).wait()
        for k in range(CH):
            ln_row(cb + k)
        return carry

    lax.fori_loop(0, bpw // CH, chunk_body, 0)

    pltpu.sync_copy(rows_v, out.at[pl.ds(base, bpw)])


def kernel(layer_indices, layer_type, L, device, emb_table, ln_gamma, ln_beta):
    del layer_type, device
    n = layer_indices.shape[0]
    assert n % NW == 0
    bpw = n // NW
    idx32 = layer_indices.astype(jnp.int32)
    vocab = emb_table.shape[0]
    table3 = emb_table.reshape(vocab // 8, 8, D)

    mesh = plsc.VectorSubcoreMesh(core_axis_name="c", subcore_axis_name="s")
    run = pl.kernel(
        functools.partial(_ln_gather_body, bpw),
        mesh=mesh,
        out_type=jax.ShapeDtypeStruct((n, D), jnp.float32),
        scratch_types=[
            pltpu.VMEM((bpw,), jnp.int32),
            pltpu.VMEM((bpw, D), jnp.float32),
            pltpu.VMEM((D,), jnp.float32),
            pltpu.VMEM((D,), jnp.float32),
            pltpu.SemaphoreType.DMA,
        ],
    )
    return run(table3, idx32, ln_gamma, ln_beta)


# speedup vs baseline: 1.9151x; 1.0035x over previous
import functools

import jax
import jax.numpy as jnp
from jax import lax
from jax.experimental import pallas as pl
from jax.experimental.pallas import tpu as pltpu
from jax.experimental.pallas import tpu_sc as plsc

D = 64
EPS = 1e-5
NC, NS, LANES = 2, 16, 16
NW = NC * NS
CH = 32


def _ln_gather_body(bpw, table3, idx, gamma, beta, out,
                    idx_v, rows_v, gam_v, bet_v, sem):
    wid = lax.axis_index("s") * NC + lax.axis_index("c")
    base = wid * bpw

    pltpu.sync_copy(idx.at[pl.ds(base, bpw)], idx_v)
    pltpu.sync_copy(gamma, gam_v)
    pltpu.sync_copy(beta, bet_v)

    def issue_body(gi, carry):
        vb = gi * LANES
        t = idx_v[pl.ds(vb, LANES)]
        gv = lax.shift_right_logical(t, 3)
        sv = lax.bitwise_and(t, 7)
        for k in range(LANES):
            pltpu.async_copy(table3.at[gv[k], sv[k]], rows_v.at[vb + k], sem)
        return carry

    lax.fori_loop(0, bpw // LANES, issue_body, 0)

    g = [gam_v[pl.ds(j * LANES, LANES)] for j in range(D // LANES)]
    b = [bet_v[pl.ds(j * LANES, LANES)] for j in range(D // LANES)]
    inv_d = jnp.float32(1.0 / D)
    iota = lax.iota(jnp.int32, LANES)
    perms = [lax.bitwise_xor(iota, jnp.int32(sh)) for sh in (8, 4, 2, 1)]

    gdn = lax.GatherDimensionNumbers(
        offset_dims=(), collapsed_slice_dims=(0,), start_index_map=(0,))

    def shuffle(x, p):
        return lax.gather(x, p[:, None], dimension_numbers=gdn,
                          slice_sizes=(1,),
                          mode=lax.GatherScatterMode.PROMISE_IN_BOUNDS)

    def allsum(x):
        for p in perms:
            x = x + shuffle(x, p)
        return x

    def ln_row(r):
        v = [rows_v[r, pl.ds(j * LANES, LANES)] for j in range(D // LANES)]
        s = (v[0] + v[1]) + (v[2] + v[3])
        sq = (v[0] * v[0] + v[1] * v[1]) + (v[2] * v[2] + v[3] * v[3])
        mean = allsum(s) * inv_d
        var = allsum(sq) * inv_d - mean * mean
        xv = var + EPS
        iv = lax.bitcast_convert_type(xv, jnp.int32)
        iv = jnp.int32(0x5F3759DF) - lax.shift_right_logical(iv, 1)
        y = lax.bitcast_convert_type(iv, jnp.float32)
        half_x = xv * jnp.float32(0.5)
        for _ in range(3):
            y = y * (jnp.float32(1.5) - half_x * y * y)
        for j in range(D // LANES):
            rows_v[r, pl.ds(j * LANES, LANES)] = (v[j] - mean) * y * g[j] + b[j]

    def chunk_body(c, carry):
        cb = c * CH
        for k in range(CH):
            pltpu.make_async_copy(
                table3.at[0, 0], rows_v.at[cb + k], sem).wait()
        for k in range(CH):
            ln_row(cb + k)
        return carry

    lax.fori_loop(0, bpw // CH, chunk_body, 0)

    pltpu.sync_copy(rows_v, out.at[pl.ds(base, bpw)])


def kernel(layer_indices, layer_type, L, device, emb_table, ln_gamma, ln_beta):
    del layer_type, device
    n = layer_indices.shape[0]
    assert n % NW == 0
    bpw = n // NW
    idx32 = layer_indices.astype(jnp.int32)
    vocab = emb_table.shape[0]
    table3 = emb_table.reshape(vocab // 8, 8, D)

    mesh = plsc.VectorSubcoreMesh(core_axis_name="c", subcore_axis_name="s")
    run = pl.kernel(
        functools.partial(_ln_gather_body, bpw),
        mesh=mesh,
        out_type=jax.ShapeDtypeStruct((n, D), jnp.float32),
        scratch_types=[
            pltpu.VMEM((bpw,), jnp.int32),
            pltpu.VMEM((bpw, D), jnp.float32),
            pltpu.VMEM((D,), jnp.float32),
            pltpu.VMEM((D,), jnp.float32),
            pltpu.SemaphoreType.DMA,
        ],
        compiler_params=pltpu.CompilerParams(use_tc_tiling_on_sc=True),
    )
    return run(table3, idx32, ln_gamma, ln_beta)
